# Initial kernel scaffold; baseline (speedup 1.0000x reference)
import jax
import jax.numpy as jnp
from jax.experimental import pallas as pl


def kernel(x, table):
    raise NotImplementedError("write your pallas kernel here")


# speedup vs baseline: 1.5551x; 1.5551x over previous
import functools

import jax
import jax.numpy as jnp
from jax import lax
from jax.experimental import pallas as pl
from jax.experimental.pallas import tpu as pltpu
from jax.experimental.pallas import tpu_sc as plsc

_D = 32
_NC = 2
_NS = 16
_NW = _NC * _NS


@functools.cache
def _make_gather(B: int, chunk: int):
    assert B % _NW == 0
    b_per_w = B // _NW
    assert b_per_w % chunk == 0
    n_chunks = b_per_w // chunk
    mesh = plsc.VectorSubcoreMesh(core_axis_name="c", subcore_axis_name="s")

    @functools.partial(
        pl.kernel,
        out_type=jax.ShapeDtypeStruct((B, _D), jnp.float32),
        mesh=mesh,
        compiler_params=pltpu.CompilerParams(use_tc_tiling_on_sc=False),
        scratch_types=[
            pltpu.VMEM((chunk,), jnp.int32),
            pltpu.VMEM((chunk, _D), jnp.float32),
            pltpu.SemaphoreType.DMA,
        ],
    )
    def gather_kernel(idx_hbm, table_hbm, out_hbm, idx_v, rows_v, sem):
        wid = lax.axis_index("s") * _NC + lax.axis_index("c")

        def body(c, carry):
            base = wid * b_per_w + c * chunk
            pltpu.sync_copy(idx_hbm.at[pl.ds(base, chunk)], idx_v)
            pltpu.async_copy(table_hbm.at[idx_v], rows_v, sem).wait()
            pltpu.sync_copy(rows_v, out_hbm.at[pl.ds(base, chunk)])
            return carry

        lax.fori_loop(0, n_chunks, body, 0)

    return gather_kernel


def kernel(x, table):
    lead_shape = x.shape
    idx = x.reshape(-1).astype(jnp.int32)
    out = _make_gather(idx.shape[0], 1024)(idx, table)
    return out.reshape(*lead_shape, _D)


# speedup vs baseline: 1.5743x; 1.0124x over previous
import functools

import jax
import jax.numpy as jnp
from jax import lax
from jax.experimental import pallas as pl
from jax.experimental.pallas import tpu as pltpu
from jax.experimental.pallas import tpu_sc as plsc

_D = 32
_NC = 2
_NS = 16
_NW = _NC * _NS
_CH = 1024
_NB = 3


@functools.cache
def _make_gather(B: int):
    assert B % (_NW * _CH) == 0
    b_per_w = B // _NW
    n_chunks = b_per_w // _CH
    mesh = plsc.VectorSubcoreMesh(core_axis_name="c", subcore_axis_name="s")

    @functools.partial(
        pl.kernel,
        out_type=jax.ShapeDtypeStruct((B, _D), jnp.float32),
        mesh=mesh,
        compiler_params=pltpu.CompilerParams(use_tc_tiling_on_sc=False),
        scratch_types=[
            pltpu.VMEM((n_chunks, _CH), jnp.int32),
            pltpu.VMEM((_NB, _CH, _D), jnp.float32),
            [pltpu.SemaphoreType.DMA] * _NB,
            [pltpu.SemaphoreType.DMA] * _NB,
        ],
    )
    def gather_kernel(idx_hbm, table_hbm, out_hbm, idx_v, rows_v, sem_g, sem_o):
        wid = lax.axis_index("s") * _NC + lax.axis_index("c")
        base_w = wid * b_per_w
        pltpu.sync_copy(idx_hbm.at[wid], idx_v)

        gathers = [None] * n_chunks
        outs = [None] * n_chunks
        for c in range(n_chunks):
            b = c % _NB
            if c >= _NB:
                outs[c - _NB].wait()
            gathers[c] = pltpu.async_copy(
                table_hbm.at[idx_v.at[c]], rows_v.at[b], sem_g[b])
            if c >= 1:
                bp = (c - 1) % _NB
                gathers[c - 1].wait()
                outs[c - 1] = pltpu.async_copy(
                    rows_v.at[bp],
                    out_hbm.at[pl.ds(base_w + (c - 1) * _CH, _CH)],
                    sem_o[bp])
        c = n_chunks - 1
        gathers[c].wait()
        outs[c] = pltpu.async_copy(
            rows_v.at[c % _NB],
            out_hbm.at[pl.ds(base_w + c * _CH, _CH)],
            sem_o[c % _NB])
        for c in range(max(0, n_chunks - _NB), n_chunks):
            outs[c].wait()

    return gather_kernel


def kernel(x, table):
    lead_shape = x.shape
    idx = x.reshape(-1).astype(jnp.int32)
    B = idx.shape[0]
    idx3 = idx.reshape(_NW, B // (_NW * _CH), _CH)
    out = _make_gather(B)(idx3, table)
    return out.reshape(*lead_shape, _D)
